# baseline (speedup 1.0000x reference)
import functools

import jax
import jax.numpy as jnp
from jax import lax
from jax.experimental import pallas as pl
from jax.experimental.pallas import tpu as pltpu
from jax.experimental.pallas import tpu_sc as plsc

_BLK = 128
_TBLK = 32768
_DS = 40


def _tc_transpose(tabT):
    dTC, V = tabT.shape
    grid = (V + _TBLK - 1) // _TBLK

    def body(i_ref, o_ref):
        o_ref[...] = i_ref[...].T

    return pl.pallas_call(
        body,
        grid=(grid,),
        in_specs=[pl.BlockSpec((dTC, _TBLK), lambda j: (0, j))],
        out_specs=pl.BlockSpec((_TBLK, dTC), lambda j: (j, 0)),
        out_shape=jax.ShapeDtypeStruct((V, dTC), jnp.float32),
    )(tabT)


def _build_gather(B, S, V, D):
    N = B * S
    DA, DB = _DS, D - _DS
    info = plsc.get_sparse_core_info()
    NC = info.num_cores
    NW = NC * info.num_subcores
    n_per_w = N // NW
    n_blocks = n_per_w // _BLK

    mesh = plsc.VectorSubcoreMesh(core_axis_name="c", subcore_axis_name="s")

    @functools.partial(
        pl.kernel,
        mesh=mesh,
        out_type=(
            jax.ShapeDtypeStruct((N, DA), jnp.float32),
            jax.ShapeDtypeStruct((N, DB), jnp.float32),
        ),
        scratch_types=[
            pltpu.VMEM((n_blocks, _BLK), jnp.int32),
            pltpu.VMEM((n_per_w, DA), jnp.float32),
            pltpu.VMEM((n_per_w, DB), jnp.float32),
            pltpu.SemaphoreType.DMA,
        ],
        compiler_params=pltpu.CompilerParams(use_tc_tiling_on_sc=False),
    )
    def emb(ids_hbm, tabA_hbm, tabB_hbm, posA_hbm, posB_hbm,
            outA_hbm, outB_hbm, idx_v, bufA_v, bufB_v, sem):
        wid = lax.axis_index("s") * NC + lax.axis_index("c")
        base = wid * n_per_w
        p0 = base % S
        pltpu.sync_copy(ids_hbm.at[pl.ds(wid * n_blocks, n_blocks)], idx_v)
        pltpu.sync_copy(posA_hbm.at[pl.ds(p0, n_per_w)], bufA_v)
        pltpu.sync_copy(posB_hbm.at[pl.ds(p0, n_per_w)], bufB_v)
        copies = []
        for j in range(n_blocks):
            idx_row = idx_v.at[j]
            copies.append(
                pltpu.async_copy(
                    tabA_hbm.at[idx_row],
                    bufA_v.at[pl.ds(j * _BLK, _BLK)],
                    sem, add=True,
                )
            )
            copies.append(
                pltpu.async_copy(
                    tabB_hbm.at[idx_row],
                    bufB_v.at[pl.ds(j * _BLK, _BLK)],
                    sem, add=True,
                )
            )
        for c in copies:
            c.wait()
        pltpu.sync_copy(bufA_v, outA_hbm.at[pl.ds(base, n_per_w)])
        pltpu.sync_copy(bufB_v, outB_hbm.at[pl.ds(base, n_per_w)])

    return emb


def kernel(token_ids, token_table, pos_table):
    B, S = token_ids.shape
    V, D = token_table.shape
    N = B * S
    ids_2d = token_ids.reshape(N // _BLK, _BLK).astype(jnp.int32)
    tabA = token_table[:, :_DS]
    tabB = _tc_transpose(token_table[:, _DS:].T)
    posA = pos_table[:, :_DS]
    posB = pos_table[:, _DS:]
    emb = _build_gather(B, S, V, D)
    outA, outB = emb(ids_2d, tabA, tabB, posA, posB)
    return jnp.concatenate([outA, outB], axis=1).reshape(B, S, D)


# speedup vs baseline: 1.0113x; 1.0113x over previous
import functools

import jax
import jax.numpy as jnp
from jax import lax
from jax.experimental import pallas as pl
from jax.experimental.pallas import tpu as pltpu
from jax.experimental.pallas import tpu_sc as plsc

_BLK = 128
_TBLK = 32768
_DS = 32


def _tc_transpose(tabT):
    dTC, V = tabT.shape
    grid = (V + _TBLK - 1) // _TBLK

    def body(i_ref, o_ref):
        o_ref[...] = i_ref[...].T

    return pl.pallas_call(
        body,
        grid=(grid,),
        in_specs=[pl.BlockSpec((dTC, _TBLK), lambda j: (0, j))],
        out_specs=pl.BlockSpec((_TBLK, dTC), lambda j: (j, 0)),
        out_shape=jax.ShapeDtypeStruct((V, dTC), jnp.float32),
    )(tabT)


def _build_gather(B, S, V, D):
    N = B * S
    DA, DB = _DS, D - _DS
    info = plsc.get_sparse_core_info()
    NC = info.num_cores
    NW = NC * info.num_subcores
    n_per_w = N // NW
    n_blocks = n_per_w // _BLK

    mesh = plsc.VectorSubcoreMesh(core_axis_name="c", subcore_axis_name="s")

    @functools.partial(
        pl.kernel,
        mesh=mesh,
        out_type=(
            jax.ShapeDtypeStruct((N, DA), jnp.float32),
            jax.ShapeDtypeStruct((N, DB), jnp.float32),
        ),
        scratch_types=[
            pltpu.VMEM((n_blocks, _BLK), jnp.int32),
            pltpu.VMEM((n_per_w, DA), jnp.float32),
            pltpu.VMEM((n_per_w, DB), jnp.float32),
            pltpu.SemaphoreType.DMA,
        ],
        compiler_params=pltpu.CompilerParams(use_tc_tiling_on_sc=False),
    )
    def emb(ids_hbm, tabA_hbm, tabB_hbm, posA_hbm, posB_hbm,
            outA_hbm, outB_hbm, idx_v, bufA_v, bufB_v, sem):
        wid = lax.axis_index("s") * NC + lax.axis_index("c")
        base = wid * n_per_w
        p0 = base % S
        pltpu.sync_copy(ids_hbm.at[pl.ds(wid * n_blocks, n_blocks)], idx_v)
        pltpu.sync_copy(posA_hbm.at[pl.ds(p0, n_per_w)], bufA_v)
        pltpu.sync_copy(posB_hbm.at[pl.ds(p0, n_per_w)], bufB_v)
        copies = []
        for j in range(n_blocks):
            idx_row = idx_v.at[j]
            copies.append(
                pltpu.async_copy(
                    tabA_hbm.at[idx_row],
                    bufA_v.at[pl.ds(j * _BLK, _BLK)],
                    sem, add=True,
                )
            )
            copies.append(
                pltpu.async_copy(
                    tabB_hbm.at[idx_row],
                    bufB_v.at[pl.ds(j * _BLK, _BLK)],
                    sem, add=True,
                )
            )
        for c in copies:
            c.wait()
        pltpu.sync_copy(bufA_v, outA_hbm.at[pl.ds(base, n_per_w)])
        pltpu.sync_copy(bufB_v, outB_hbm.at[pl.ds(base, n_per_w)])

    return emb


def kernel(token_ids, token_table, pos_table):
    B, S = token_ids.shape
    V, D = token_table.shape
    N = B * S
    ids_2d = token_ids.reshape(N // _BLK, _BLK).astype(jnp.int32)
    tabA = token_table[:, :_DS]
    tabB = _tc_transpose(token_table[:, _DS:].T)
    posA = pos_table[:, :_DS]
    posB = pos_table[:, _DS:]
    emb = _build_gather(B, S, V, D)
    outA, outB = emb(ids_2d, tabA, tabB, posA, posB)
    return jnp.concatenate([outA, outB], axis=1).reshape(B, S, D)


# speedup vs baseline: 4.2953x; 4.2474x over previous
import functools

import jax
import jax.numpy as jnp
from jax import lax
from jax.experimental import pallas as pl
from jax.experimental.pallas import tpu as pltpu
from jax.experimental.pallas import tpu_sc as plsc

_BLK = 128
_CBLK = 601


def _tc_detile(tabT):
    D, V = tabT.shape
    ntiles = (V + 127) // 128
    ncblk = (ntiles + _CBLK - 1) // _CBLK
    gg = D // 8

    def body(i_ref, o_ref):
        x = i_ref[...]
        x = x.reshape(8, _CBLK, 128)
        o_ref[...] = x.transpose(1, 0, 2).reshape(_CBLK * 8, 128)

    return pl.pallas_call(
        body,
        grid=(gg, ncblk),
        in_specs=[pl.BlockSpec((8, _CBLK * 128), lambda g, j: (g, j))],
        out_specs=pl.BlockSpec((_CBLK * 8, 128), lambda g, j: (g * ncblk + j, 0)),
        out_shape=jax.ShapeDtypeStruct((gg * ncblk * _CBLK * 8, 128), jnp.float32),
    )(tabT)


def _build_gather(B, S, V, D):
    N = B * S
    info = plsc.get_sparse_core_info()
    NC = info.num_cores
    NL = info.num_lanes
    NW = NC * info.num_subcores
    n_per_w = N // NW
    n_blocks = n_per_w // _BLK
    DGRP = 16
    ntiles = (V + 127) // 128
    gstride = ntiles * 1024
    gsize = (ntiles - 1) * 1024 + 128

    mesh = plsc.VectorSubcoreMesh(core_axis_name="c", subcore_axis_name="s")

    @functools.partial(
        pl.kernel,
        mesh=mesh,
        out_type=jax.ShapeDtypeStruct((N * D,), jnp.float32),
        scratch_types=[
            pltpu.VMEM((n_per_w,), jnp.int32),
            pltpu.VMEM((n_per_w,), jnp.int32),
            pltpu.VMEM((D * n_per_w,), jnp.float32),
            pltpu.VMEM((DGRP * n_per_w,), jnp.float32),
            pltpu.SemaphoreType.DMA,
        ],
        compiler_params=pltpu.CompilerParams(use_tc_tiling_on_sc=False),
    )
    def emb(ids_hbm, img_hbm, posT_hbm, out_hbm, ids_v, base_v, buf_v, pos_v, sem):
        wid = lax.axis_index("s") * NC + lax.axis_index("c")
        b = wid // (S // n_per_w)
        s0 = (wid % (S // n_per_w)) * n_per_w
        pltpu.sync_copy(ids_hbm.at[pl.ds(wid * n_per_w, n_per_w)], ids_v)

        @pl.loop(0, n_per_w // NL)
        def _(i):
            t = ids_v[pl.ds(i * NL, NL)]
            base_v[pl.ds(i * NL, NL)] = (
                lax.shift_left(lax.shift_right_logical(t, 7), 10)
                + lax.bitwise_and(t, 127)
            )

        for q in range(D // DGRP):

            @pl.loop(0, n_blocks)
            def _(j, _q=q):
                idx_row = base_v.at[pl.ds(j * _BLK, _BLK)]
                copies = []
                for k in range(DGRP):
                    d = _q * DGRP + k
                    off = (d // 8) * gstride + (d % 8) * 128
                    copies.append(
                        pltpu.async_copy(
                            img_hbm.at[pl.ds(off, gsize)].at[idx_row],
                            buf_v.at[pl.ds(d * n_per_w + j * _BLK, _BLK)],
                            sem,
                        )
                    )
                for c in copies:
                    c.wait()

        for q in range(D // DGRP):
            seeds = [
                pltpu.async_copy(
                    posT_hbm.at[pl.ds((q * DGRP + k) * S + s0, n_per_w)],
                    pos_v.at[pl.ds(k * n_per_w, n_per_w)],
                    sem,
                )
                for k in range(DGRP)
            ]
            for c in seeds:
                c.wait()

            @pl.loop(0, DGRP)
            def _(k, _q=q):
                base = (_q * DGRP) * n_per_w + k * n_per_w
                pbase = k * n_per_w
                for r in range(n_per_w // NL):
                    o = r * NL
                    buf_v[pl.ds(base + o, NL)] = (
                        buf_v[pl.ds(base + o, NL)] + pos_v[pl.ds(pbase + o, NL)]
                    )

        obase = b * (D * S) + s0
        for q in range(D // DGRP):
            outs = [
                pltpu.async_copy(
                    buf_v.at[pl.ds((q * DGRP + k) * n_per_w, n_per_w)],
                    out_hbm.at[pl.ds(obase + (q * DGRP + k) * S, n_per_w)],
                    sem,
                )
                for k in range(DGRP)
            ]
            for c in outs:
                c.wait()

    return emb


def kernel(token_ids, token_table, pos_table):
    B, S = token_ids.shape
    V, D = token_table.shape
    N = B * S
    ids_flat = token_ids.reshape(N).astype(jnp.int32)
    img = _tc_detile(token_table.T)
    img_flat = img.reshape(img.shape[0] * 128)
    posT = pos_table.T.reshape(D * S)
    emb = _build_gather(B, S, V, D)
    out = emb(ids_flat, img_flat, posT)
    return jnp.swapaxes(out.reshape(B, D, S), 1, 2)


# speedup vs baseline: 4.4392x; 1.0335x over previous
import functools

import jax
import jax.numpy as jnp
from jax import lax
from jax.experimental import pallas as pl
from jax.experimental.pallas import tpu as pltpu
from jax.experimental.pallas import tpu_sc as plsc

_BLK = 128
_CBLK = 601


def _tc_detile(tabT):
    D, V = tabT.shape
    ntiles = (V + 127) // 128
    ncblk = (ntiles + _CBLK - 1) // _CBLK
    gg = D // 8

    def body(i_ref, o_ref):
        x = i_ref[...]
        x = x.reshape(8, _CBLK, 128)
        o_ref[...] = x.transpose(1, 0, 2).reshape(_CBLK * 8, 128)

    return pl.pallas_call(
        body,
        grid=(gg, ncblk),
        in_specs=[pl.BlockSpec((8, _CBLK * 128), lambda g, j: (g, j))],
        out_specs=pl.BlockSpec((_CBLK * 8, 128), lambda g, j: (g * ncblk + j, 0)),
        out_shape=jax.ShapeDtypeStruct((gg * ncblk * _CBLK * 8, 128), jnp.float32),
    )(tabT)


def _build_gather(B, S, V, D):
    N = B * S
    info = plsc.get_sparse_core_info()
    NC = info.num_cores
    NL = info.num_lanes
    NW = NC * info.num_subcores
    n_per_w = N // NW
    n_blocks = n_per_w // _BLK
    DGRP = 32
    ntiles = (V + 127) // 128
    gstride = ntiles * 1024
    gsize = (ntiles - 1) * 1024 + 128

    mesh = plsc.VectorSubcoreMesh(core_axis_name="c", subcore_axis_name="s")

    @functools.partial(
        pl.kernel,
        mesh=mesh,
        out_type=jax.ShapeDtypeStruct((N * D,), jnp.float32),
        scratch_types=[
            pltpu.VMEM((n_per_w,), jnp.int32),
            pltpu.VMEM((n_per_w,), jnp.int32),
            pltpu.VMEM((D * n_per_w,), jnp.float32),
            pltpu.VMEM((DGRP * n_per_w,), jnp.float32),
            pltpu.SemaphoreType.DMA,
        ],
        compiler_params=pltpu.CompilerParams(use_tc_tiling_on_sc=False),
    )
    def emb(ids_hbm, img_hbm, posT_hbm, out_hbm, ids_v, base_v, buf_v, pos_v, sem):
        wid = lax.axis_index("s") * NC + lax.axis_index("c")
        b = wid // (S // n_per_w)
        s0 = (wid % (S // n_per_w)) * n_per_w
        pltpu.sync_copy(ids_hbm.at[pl.ds(wid * n_per_w, n_per_w)], ids_v)

        @pl.loop(0, n_per_w // NL)
        def _(i):
            t = ids_v[pl.ds(i * NL, NL)]
            base_v[pl.ds(i * NL, NL)] = (
                lax.shift_left(lax.shift_right_logical(t, 7), 10)
                + lax.bitwise_and(t, 127)
            )

        for q in range(D // DGRP):

            @pl.loop(0, n_blocks)
            def _(j, _q=q):
                idx_row = base_v.at[pl.ds(j * _BLK, _BLK)]
                copies = []
                for k in range(DGRP):
                    d = _q * DGRP + k
                    off = (d // 8) * gstride + (d % 8) * 128
                    copies.append(
                        pltpu.async_copy(
                            img_hbm.at[pl.ds(off, gsize)].at[idx_row],
                            buf_v.at[pl.ds(d * n_per_w + j * _BLK, _BLK)],
                            sem,
                        )
                    )
                for c in copies:
                    c.wait()

        for q in range(D // DGRP):
            seeds = [
                pltpu.async_copy(
                    posT_hbm.at[pl.ds((q * DGRP + k) * S + s0, n_per_w)],
                    pos_v.at[pl.ds(k * n_per_w, n_per_w)],
                    sem,
                )
                for k in range(DGRP)
            ]
            for c in seeds:
                c.wait()

            @pl.loop(0, DGRP)
            def _(k, _q=q):
                base = (_q * DGRP) * n_per_w + k * n_per_w
                pbase = k * n_per_w
                for r in range(n_per_w // NL):
                    o = r * NL
                    buf_v[pl.ds(base + o, NL)] = (
                        buf_v[pl.ds(base + o, NL)] + pos_v[pl.ds(pbase + o, NL)]
                    )

        obase = b * (D * S) + s0
        for q in range(D // DGRP):
            outs = [
                pltpu.async_copy(
                    buf_v.at[pl.ds((q * DGRP + k) * n_per_w, n_per_w)],
                    out_hbm.at[pl.ds(obase + (q * DGRP + k) * S, n_per_w)],
                    sem,
                )
                for k in range(DGRP)
            ]
            for c in outs:
                c.wait()

    return emb


def kernel(token_ids, token_table, pos_table):
    B, S = token_ids.shape
    V, D = token_table.shape
    N = B * S
    ids_flat = token_ids.reshape(N).astype(jnp.int32)
    img = _tc_detile(token_table.T)
    img_flat = img.reshape(img.shape[0] * 128)
    posT = pos_table.T.reshape(D * S)
    emb = _build_gather(B, S, V, D)
    out = emb(ids_flat, img_flat, posT)
    return jnp.swapaxes(out.reshape(B, D, S), 1, 2)
